# Initial kernel scaffold; baseline (speedup 1.0000x reference)
import jax
import jax.numpy as jnp
from jax.experimental import pallas as pl


def kernel(x, edge_index, edge_weight, W1, b1, W2, b2, W3, b3):
    raise NotImplementedError("write your pallas kernel here")


# speedup vs baseline: 8.8691x; 8.8691x over previous
import functools

import jax
import jax.numpy as jnp
from jax import lax
from jax.experimental import pallas as pl
from jax.experimental.pallas import tpu as pltpu
from jax.experimental.pallas import tpu_sc as plsc

N_NODES = 10000
D_HID = 128

NC = 2
NS = 16
NW = NC * NS

CH = 128
E_EDGES = 320000
PER_W = 10112
NCHUNK = PER_W // CH
E_PAD = NW * PER_W

N_DEG_PAD = 10240
N_ACC_PAD = 10112


def _deg_body(dst_hbm, ew_hbm, out_hbm, deg_sp, dstv, ewv, zbuf):
    cid = lax.axis_index("c")
    sid = lax.axis_index("s")
    wid = sid * NC + cid

    def zb(i, _):
        zbuf[pl.ds(i * 16, 16)] = jnp.zeros((16,), jnp.float32)
        return 0

    lax.fori_loop(0, 40, zb, 0)
    pltpu.sync_copy(zbuf, deg_sp.at[pl.ds(sid * 640, 640)])
    plsc.subcore_barrier()

    base0 = wid * PER_W

    def chunk(g, _):
        base = base0 + g * CH
        pltpu.sync_copy(dst_hbm.at[pl.ds(base, CH)], dstv)
        pltpu.sync_copy(ew_hbm.at[pl.ds(base, CH)], ewv)
        pltpu.sync_copy(ewv, deg_sp.at[dstv], add=True)
        return 0

    lax.fori_loop(0, NCHUNK, chunk, 0)
    plsc.subcore_barrier()
    pltpu.sync_copy(deg_sp.at[pl.ds(sid * 640, 640)],
                    out_hbm.at[cid, pl.ds(sid * 640, 640)])


def _spmm_body(hp_hbm, src_hbm, dst_hbm, ew_hbm, out_hbm,
               acc_sp, srcv, dstv, ewv, rows, sem):
    cid = lax.axis_index("c")
    sid = lax.axis_index("s")
    wid = sid * NC + cid

    def zb(i, _):
        for j in range(8):
            rows[i, pl.ds(j * 16, 16)] = jnp.zeros((16,), jnp.float32)
        return 0

    lax.fori_loop(0, CH, zb, 0)
    for k in range(4):
        pltpu.sync_copy(rows, acc_sp.at[pl.ds(sid * 632 + k * CH, CH)])
    pltpu.sync_copy(rows.at[pl.ds(0, 120)],
                    acc_sp.at[pl.ds(sid * 632 + 4 * CH, 120)])
    plsc.subcore_barrier()

    base0 = wid * PER_W

    def chunk(g, _):
        base = base0 + g * CH
        pltpu.sync_copy(src_hbm.at[pl.ds(base, CH)], srcv)
        pltpu.sync_copy(dst_hbm.at[pl.ds(base, CH)], dstv)
        pltpu.sync_copy(ew_hbm.at[pl.ds(base, CH)], ewv)
        pltpu.async_copy(hp_hbm.at[srcv], rows, sem).wait()

        def scale(g, _):
            ew16 = ewv[pl.ds(g * 16, 16)]
            for i in range(16):
                s = ew16[i]
                r = g * 16 + i
                for j in range(8):
                    sl = pl.ds(j * 16, 16)
                    rows[r, sl] = rows[r, sl] * s
            return 0

        lax.fori_loop(0, CH // 16, scale, 0)
        pltpu.sync_copy(rows, acc_sp.at[dstv], add=True)
        return 0

    lax.fori_loop(0, NCHUNK, chunk, 0)
    plsc.subcore_barrier()
    off = sid * 632
    pltpu.sync_copy(acc_sp.at[pl.ds(off, 632)],
                    out_hbm.at[cid, pl.ds(off, 632)])


_deg_kernel = functools.partial(
    pl.kernel,
    _deg_body,
    out_type=jax.ShapeDtypeStruct((NC, N_DEG_PAD), jnp.float32),
    mesh=plsc.VectorSubcoreMesh(core_axis_name="c", subcore_axis_name="s"),
    scratch_types=[
        pltpu.VMEM_SHARED((N_DEG_PAD,), jnp.float32),
        pltpu.VMEM((CH,), jnp.int32),
        pltpu.VMEM((CH,), jnp.float32),
        pltpu.VMEM((640,), jnp.float32),
    ],
)()

_spmm_kernel = functools.partial(
    pl.kernel,
    _spmm_body,
    out_type=jax.ShapeDtypeStruct((NC, N_ACC_PAD, D_HID), jnp.float32),
    mesh=plsc.VectorSubcoreMesh(core_axis_name="c", subcore_axis_name="s"),
    scratch_types=[
        pltpu.VMEM_SHARED((N_ACC_PAD, D_HID), jnp.float32),
        pltpu.VMEM((CH,), jnp.int32),
        pltpu.VMEM((CH,), jnp.int32),
        pltpu.VMEM((CH,), jnp.float32),
        pltpu.VMEM((CH, D_HID), jnp.float32),
        pltpu.SemaphoreType.DMA,
    ],
)()


def _tcb_body(x_ref, w_ref, degp_ref, hp_ref, dinv_ref):
    deg = degp_ref[0] + degp_ref[1] + 1.0
    dinv = jnp.where(deg > 0, lax.rsqrt(deg), 0.0)
    dinv_ref[...] = dinv
    hp_ref[...] = jnp.dot(x_ref[...], w_ref[...],
                          preferred_element_type=jnp.float32) * dinv


def _tcd_body(acc_ref, hp_ref, dinv_ref, w_ref, b_ref, out_ref):
    dinv = dinv_ref[...]
    acc = acc_ref[0, :N_NODES, :] + acc_ref[1, :N_NODES, :]
    pre = dinv * (acc + hp_ref[...]) + b_ref[...]
    h = jnp.maximum(pre, 0.0)
    out_ref[...] = jnp.dot(h, w_ref[...],
                           preferred_element_type=jnp.float32) * dinv


def _tcf_body(acc_ref, hp_ref, dinv_ref, w_ref, b_ref, b3_ref, out_ref):
    dinv = dinv_ref[...]
    acc = acc_ref[0, :N_NODES, :] + acc_ref[1, :N_NODES, :]
    pre = dinv * (acc + hp_ref[...]) + b_ref[...]
    h = jnp.maximum(pre, 0.0)
    out_ref[...] = jnp.dot(h, w_ref[...],
                           preferred_element_type=jnp.float32) + b3_ref[...]


def kernel(x, edge_index, edge_weight, W1, b1, W2, b2, W3, b3):
    src = edge_index[0].astype(jnp.int32)
    dst = edge_index[1].astype(jnp.int32)
    ew = edge_weight.astype(jnp.float32)

    pad = E_PAD - E_EDGES
    zi = jnp.zeros((pad,), jnp.int32)
    src_p = jnp.concatenate([src, zi])
    dst_p = jnp.concatenate([dst, zi])
    ew_p = jnp.concatenate([ew, jnp.zeros((pad,), jnp.float32)])

    deg_parts = _deg_kernel(dst_p, ew_p)
    degp = deg_parts[:, :N_NODES, None]

    hp1, dinv = pl.pallas_call(
        _tcb_body,
        out_shape=(
            jax.ShapeDtypeStruct((N_NODES, D_HID), jnp.float32),
            jax.ShapeDtypeStruct((N_NODES, 1), jnp.float32),
        ),
    )(x, W1, degp)

    acc1 = _spmm_kernel(hp1, src_p, dst_p, ew_p)

    hp2 = pl.pallas_call(
        _tcd_body,
        out_shape=jax.ShapeDtypeStruct((N_NODES, D_HID), jnp.float32),
    )(acc1, hp1, dinv, W2, b1[None, :])

    acc2 = _spmm_kernel(hp2, src_p, dst_p, ew_p)

    out = pl.pallas_call(
        _tcf_body,
        out_shape=jax.ShapeDtypeStruct((N_NODES, W3.shape[1]), jnp.float32),
    )(acc2, hp2, dinv, W3, b2[None, :], b3[None, :])

    return out
